# Initial kernel scaffold; baseline (speedup 1.0000x reference)
import jax
import jax.numpy as jnp
from jax.experimental import pallas as pl


def kernel(x, table):
    raise NotImplementedError("write your pallas kernel here")


# speedup vs baseline: 1.6360x; 1.6360x over previous
import functools

import jax
import jax.numpy as jnp
from jax import lax
from jax.experimental import pallas as pl
from jax.experimental.pallas import tpu as pltpu
from jax.experimental.pallas import tpu_sc as plsc

NC = 2
NS = 16
NW = NC * NS
LANES = 16

VOCAB_DIM = 300
DP = 304
BATCH = 16384
PER_W = BATCH // NW
CHUNK = 64
NCHUNK = PER_W // CHUNK
NSLICE = DP // LANES


def _sc_body(x0_hbm, x1_hbm, tbl_hbm, out_hbm, idx0_v, idx1_v,
             ra_v, rb_v, acc_v, sem0, sem1, semg):
    wid = lax.axis_index("s") * NC + lax.axis_index("c")
    base = wid * PER_W

    ca = pltpu.async_copy(x0_hbm.at[pl.ds(base, PER_W)], idx0_v, sem0)
    cb = pltpu.async_copy(x1_hbm.at[pl.ds(base, PER_W)], idx1_v, sem1)
    ca.wait()
    cb.wait()

    def chunk_step(g, acc):
        ga = pltpu.async_copy(
            tbl_hbm.at[idx0_v.at[pl.ds(g * CHUNK, CHUNK)]], ra_v, sem0)
        gb = pltpu.async_copy(
            tbl_hbm.at[idx1_v.at[pl.ds(g * CHUNK, CHUNK)]], rb_v, sem1)
        ga.wait()
        gb.wait()

        def row_step(r, acc):
            for j in range(NSLICE):
                a = ra_v[r, pl.ds(j * LANES, LANES)]
                b = rb_v[r, pl.ds(j * LANES, LANES)]
                acc = acc + a * b
            return acc

        return lax.fori_loop(0, CHUNK, row_step, acc)

    acc = lax.fori_loop(0, NCHUNK, chunk_step,
                        jnp.zeros((LANES,), jnp.float32))
    acc_v[...] = acc
    pltpu.async_copy(acc_v, out_hbm.at[wid], semg).wait()


@jax.jit
def _sc_gather_dot(x0, x1, table_p):
    mesh = plsc.VectorSubcoreMesh(core_axis_name="c", subcore_axis_name="s")
    return pl.kernel(
        _sc_body,
        out_type=jax.ShapeDtypeStruct((NW, LANES), jnp.float32),
        mesh=mesh,
        compiler_params=pltpu.CompilerParams(use_tc_tiling_on_sc=False),
        scratch_types=[
            pltpu.VMEM((PER_W,), jnp.int32),
            pltpu.VMEM((PER_W,), jnp.int32),
            pltpu.VMEM((CHUNK, DP), jnp.float32),
            pltpu.VMEM((CHUNK, DP), jnp.float32),
            pltpu.VMEM((LANES,), jnp.float32),
            pltpu.SemaphoreType.DMA,
            pltpu.SemaphoreType.DMA,
            pltpu.SemaphoreType.DMA,
        ],
    )(x0, x1, table_p)


def _finish_body(p_ref, o_ref):
    s = jnp.sum(p_ref[...])
    o_ref[0, 0] = s * s


@jax.jit
def _finish(partials):
    out = pl.pallas_call(
        _finish_body,
        out_shape=jax.ShapeDtypeStruct((1, 1), jnp.float32),
        out_specs=pl.BlockSpec(memory_space=pltpu.SMEM),
    )(partials)
    return out[0, 0]


def kernel(x, table):
    x0 = x[:, 0]
    x1 = x[:, 1]
    table_p = jnp.pad(table, ((0, 0), (0, DP - VOCAB_DIM)))
    partials = _sc_gather_dot(x0, x1, table_p)
    return _finish(partials)


# speedup vs baseline: 4.4029x; 2.6914x over previous
import functools

import jax
import jax.numpy as jnp
from jax import lax
from jax.experimental import pallas as pl
from jax.experimental.pallas import tpu as pltpu
from jax.experimental.pallas import tpu_sc as plsc

NC = 2
NS = 16
NW = NC * NS
LANES = 16

VOCAB_DIM = 300
MAIN = 256
TW = 128
NTS = 3
BATCH = 16384
PER_W = BATCH // NW
CHUNK = 128
NCHUNK = PER_W // CHUNK


def _sc_body(x0_hbm, x1_hbm, tbl_hbm, ttl_hbm, out_hbm,
             idx0_v, idx1_v, ma_v, mb_v, ta_v, tb_v, stage_v,
             s0, s1, s2, s3, sg):
    wid = lax.axis_index("s") * NC + lax.axis_index("c")
    base = wid * PER_W
    ca = pltpu.async_copy(x0_hbm.at[pl.ds(base, PER_W)], idx0_v, s0)
    cb = pltpu.async_copy(x1_hbm.at[pl.ds(base, PER_W)], idx1_v, s1)
    ca.wait()
    cb.wait()

    def chunk_step(g, acc):
        i0 = idx0_v.at[pl.ds(g * CHUNK, CHUNK)]
        i1 = idx1_v.at[pl.ds(g * CHUNK, CHUNK)]
        g0 = pltpu.async_copy(tbl_hbm.at[i0, pl.ds(0, MAIN)], ma_v, s0)
        g1 = pltpu.async_copy(tbl_hbm.at[i1, pl.ds(0, MAIN)], mb_v, s1)
        g2 = pltpu.async_copy(ttl_hbm.at[i0], ta_v, s2)
        g3 = pltpu.async_copy(ttl_hbm.at[i1], tb_v, s3)
        g0.wait()
        g1.wait()
        g2.wait()
        g3.wait()

        def row_step(r, acc):
            for j in range(MAIN // LANES):
                a = ma_v[r, pl.ds(j * LANES, LANES)]
                b = mb_v[r, pl.ds(j * LANES, LANES)]
                acc = acc + a * b
            for j in range(NTS):
                a = ta_v[r, pl.ds(j * LANES, LANES)]
                b = tb_v[r, pl.ds(j * LANES, LANES)]
                acc = acc + a * b
            return acc

        return lax.fori_loop(0, CHUNK, row_step, acc)

    acc = lax.fori_loop(0, NCHUNK, chunk_step,
                        jnp.zeros((LANES,), jnp.float32))

    for i in range(8):
        for j in range(128 // LANES):
            stage_v[i, pl.ds(j * LANES, LANES)] = jnp.zeros(
                (LANES,), jnp.float32)
    stage_v[0, pl.ds(0, LANES)] = acc
    pltpu.async_copy(stage_v, out_hbm.at[wid], sg).wait()


@jax.jit
def _sc_gather_dot(x0, x1, table, tail_tbl):
    mesh = plsc.VectorSubcoreMesh(core_axis_name="c", subcore_axis_name="s")
    return pl.kernel(
        _sc_body,
        out_type=jax.ShapeDtypeStruct((NW, 8, 128), jnp.float32),
        mesh=mesh,
        scratch_types=[
            pltpu.VMEM((PER_W,), jnp.int32),
            pltpu.VMEM((PER_W,), jnp.int32),
            pltpu.VMEM((CHUNK, MAIN), jnp.float32),
            pltpu.VMEM((CHUNK, MAIN), jnp.float32),
            pltpu.VMEM((CHUNK, TW), jnp.float32),
            pltpu.VMEM((CHUNK, TW), jnp.float32),
            pltpu.VMEM((8, 128), jnp.float32),
            pltpu.SemaphoreType.DMA,
            pltpu.SemaphoreType.DMA,
            pltpu.SemaphoreType.DMA,
            pltpu.SemaphoreType.DMA,
            pltpu.SemaphoreType.DMA,
        ],
    )(x0, x1, table, tail_tbl)


def _finish_body(p_ref, o_ref):
    s = jnp.sum(p_ref[...])
    o_ref[0, 0] = s * s


@jax.jit
def _finish(partials):
    out = pl.pallas_call(
        _finish_body,
        out_shape=jax.ShapeDtypeStruct((1, 1), jnp.float32),
        out_specs=pl.BlockSpec(memory_space=pltpu.SMEM),
    )(partials)
    return out[0, 0]


def kernel(x, table):
    x0 = x[:, 0]
    x1 = x[:, 1]
    tail_tbl = jnp.pad(table[:, MAIN:],
                       ((0, 0), (0, TW - (VOCAB_DIM - MAIN))))
    partials = _sc_gather_dot(x0, x1, table, tail_tbl)
    return _finish(partials)


# speedup vs baseline: 4.8244x; 1.0957x over previous
import functools

import jax
import jax.numpy as jnp
from jax import lax
from jax.experimental import pallas as pl
from jax.experimental.pallas import tpu as pltpu
from jax.experimental.pallas import tpu_sc as plsc

NC = 2
NS = 16
NW = NC * NS
LANES = 16

VOCAB_DIM = 300
MAIN = 256
TW = 128
NTS = 3
BATCH = 16384
PER_W = BATCH // NW
CHUNK = 64
NCHUNK = PER_W // CHUNK
NBUF = 2


def _sc_body(x0_hbm, x1_hbm, tbl_hbm, ttl_hbm, out_hbm,
             idx0_v, idx1_v,
             ma0_v, mb0_v, ta0_v, tb0_v,
             ma1_v, mb1_v, ta1_v, tb1_v,
             stage_v, s0, s1, sg):
    wid = lax.axis_index("s") * NC + lax.axis_index("c")
    base = wid * PER_W
    ca = pltpu.async_copy(x0_hbm.at[pl.ds(base, PER_W)], idx0_v, s0)
    cb = pltpu.async_copy(x1_hbm.at[pl.ds(base, PER_W)], idx1_v, s1)
    ca.wait()
    cb.wait()

    bufs = [(ma0_v, mb0_v, ta0_v, tb0_v), (ma1_v, mb1_v, ta1_v, tb1_v)]
    sems = [s0, s1]

    def issue(g):
        slot = g % NBUF
        i0 = idx0_v.at[pl.ds(g * CHUNK, CHUNK)]
        i1 = idx1_v.at[pl.ds(g * CHUNK, CHUNK)]
        ma, mb, ta, tb = bufs[slot]
        sem = sems[slot]
        return (
            pltpu.async_copy(tbl_hbm.at[i0, pl.ds(0, MAIN)], ma, sem),
            pltpu.async_copy(tbl_hbm.at[i1, pl.ds(0, MAIN)], mb, sem),
            pltpu.async_copy(ttl_hbm.at[i0], ta, sem),
            pltpu.async_copy(ttl_hbm.at[i1], tb, sem),
        )

    acc = jnp.zeros((LANES,), jnp.float32)
    pending = {}
    for g in range(min(NBUF, NCHUNK)):
        pending[g] = issue(g)
    for g in range(NCHUNK):
        slot = g % NBUF
        for h in pending.pop(g):
            h.wait()
        ma, mb, ta, tb = bufs[slot]

        def row_step(r, acc, ma=ma, mb=mb, ta=ta, tb=tb):
            for j in range(MAIN // LANES):
                a = ma[r, pl.ds(j * LANES, LANES)]
                b = mb[r, pl.ds(j * LANES, LANES)]
                acc = acc + a * b
            for j in range(NTS):
                a = ta[r, pl.ds(j * LANES, LANES)]
                b = tb[r, pl.ds(j * LANES, LANES)]
                acc = acc + a * b
            return acc

        acc = lax.fori_loop(0, CHUNK, row_step, acc)
        if g + NBUF < NCHUNK:
            pending[g + NBUF] = issue(g + NBUF)

    for i in range(8):
        for j in range(128 // LANES):
            stage_v[i, pl.ds(j * LANES, LANES)] = jnp.zeros(
                (LANES,), jnp.float32)
    stage_v[0, pl.ds(0, LANES)] = acc
    pltpu.async_copy(stage_v, out_hbm.at[wid], sg).wait()


@jax.jit
def _sc_gather_dot(x0, x1, table, tail_tbl):
    mesh = plsc.VectorSubcoreMesh(core_axis_name="c", subcore_axis_name="s")
    return pl.kernel(
        _sc_body,
        out_type=jax.ShapeDtypeStruct((NW, 8, 128), jnp.float32),
        mesh=mesh,
        scratch_types=[
            pltpu.VMEM((PER_W,), jnp.int32),
            pltpu.VMEM((PER_W,), jnp.int32),
            pltpu.VMEM((CHUNK, MAIN), jnp.float32),
            pltpu.VMEM((CHUNK, MAIN), jnp.float32),
            pltpu.VMEM((CHUNK, TW), jnp.float32),
            pltpu.VMEM((CHUNK, TW), jnp.float32),
            pltpu.VMEM((CHUNK, MAIN), jnp.float32),
            pltpu.VMEM((CHUNK, MAIN), jnp.float32),
            pltpu.VMEM((CHUNK, TW), jnp.float32),
            pltpu.VMEM((CHUNK, TW), jnp.float32),
            pltpu.VMEM((8, 128), jnp.float32),
            pltpu.SemaphoreType.DMA,
            pltpu.SemaphoreType.DMA,
            pltpu.SemaphoreType.DMA,
        ],
    )(x0, x1, table, tail_tbl)


def _finish_body(p_ref, o_ref):
    s = jnp.sum(p_ref[...])
    o_ref[0, 0] = s * s


@jax.jit
def _finish(partials):
    out = pl.pallas_call(
        _finish_body,
        out_shape=jax.ShapeDtypeStruct((1, 1), jnp.float32),
        out_specs=pl.BlockSpec(memory_space=pltpu.SMEM),
    )(partials)
    return out[0, 0]


def kernel(x, table):
    x0 = x[:, 0]
    x1 = x[:, 1]
    tail_tbl = jnp.pad(table[:, MAIN:],
                       ((0, 0), (0, TW - (VOCAB_DIM - MAIN))))
    partials = _sc_gather_dot(x0, x1, table, tail_tbl)
    return _finish(partials)
